# baseline (speedup 1.0000x reference)
import functools

import jax
import jax.numpy as jnp
from jax import lax
from jax.experimental import pallas as pl
from jax.experimental.pallas import tpu as pltpu
from jax.experimental.pallas import tpu_sc as plsc

_NC = 2
_NS = 16
_NW = _NC * _NS
_CH = 128
_NBUF = 4
_HD = 2
_L = 16


def _make_gather(n, d, n_ch):
    mesh = plsc.VectorSubcoreMesh(core_axis_name="c", subcore_axis_name="s")
    rows_per_w = n // _NW
    orows_per_w = rows_per_w * d // 128
    och = _CH * d // 128

    @functools.partial(
        pl.kernel,
        mesh=mesh,
        compiler_params=pltpu.CompilerParams(needs_layout_passes=False),
        out_type=jax.ShapeDtypeStruct((n * d // 128, 128), jnp.float32),
        scratch_types=[
            pltpu.VMEM((n_ch, _CH), jnp.int32),
            pltpu.VMEM((_NBUF, _CH), jnp.int32),
            pltpu.VMEM((_NBUF, _CH, 128), jnp.float32),
            pltpu.VMEM((2, _CH * d // 128, 128), jnp.float32),
            pltpu.SemaphoreType.DMA,
            pltpu.SemaphoreType.DMA,
        ],
    )
    def gather_kernel(x_hbm, table_hbm, out_hbm, idx_v, idxq_v, rows_v, out_v,
                      gsem, ssem):
        wid = lax.axis_index("s") * _NC + lax.axis_index("c")
        obase = wid * orows_per_w
        pltpu.sync_copy(x_hbm.at[wid], idx_v)

        lane = lax.iota(jnp.int32, _L)

        def issue_gather(chunk, slot):
            for g in range(_CH // _L):
                sl = pl.ds(g * _L, _L)
                idxq_v[slot, sl] = lax.shift_right_logical(idx_v[chunk, sl], 2)
            pltpu.async_copy(table_hbm.at[idxq_v.at[slot]], rows_v.at[slot],
                             gsem)

        def wait_gather(slot):
            pltpu.make_async_copy(
                table_hbm.at[idxq_v.at[0]], rows_v.at[slot], gsem
            ).wait()

        def extract(chunk, slot, oslot):
            rows2d = rows_v.at[slot]
            out2d = out_v.at[oslot]
            lane_d = lane * d
            for g in range(_CH // _L):
                sl = pl.ds(g * _L, _L)
                row_g = lane + g * _L
                qcol = lax.shift_left(
                    lax.bitwise_and(idx_v[chunk, sl], 3), 5)
                for j in range(d):
                    vals = plsc.load_gather(rows2d, [row_g, qcol + j])
                    flat = lane_d + (g * _L * d + j)
                    plsc.store_scatter(
                        out2d,
                        [lax.shift_right_logical(flat, 7),
                         lax.bitwise_and(flat, 127)],
                        vals)

        def issue_scatter(chunk, oslot):
            pltpu.async_copy(
                out_v.at[oslot],
                out_hbm.at[pl.ds(obase + chunk * och, och)], ssem)

        def wait_scatter(oslot):
            pltpu.make_async_copy(
                out_v.at[oslot],
                out_hbm.at[pl.ds(obase, och)], ssem
            ).wait()

        for i in range(_HD):
            issue_gather(i, i)

        def body(i, carry):
            ip = i + _HD
            bp = lax.rem(ip, _NBUF)
            b = lax.rem(i, _NBUF)
            oslot = lax.rem(i, 2)

            @pl.when(ip < n_ch)
            def _():
                issue_gather(ip, bp)

            wait_gather(b)

            @pl.when(i >= 2)
            def _():
                wait_scatter(oslot)

            extract(i, b, oslot)
            issue_scatter(i, oslot)
            return carry

        lax.fori_loop(0, n_ch, body, 0)
        wait_scatter(0)
        wait_scatter(1)

    return gather_kernel


def kernel(x, table):
    b0, s = x.shape
    v, d = table.shape
    n = b0 * s
    n_ch = n // (_NW * _CH)
    x_blocked = x.reshape(_NW, n_ch, _CH)
    table_wide = table.reshape(v * d // 128, 128)
    out = _make_gather(n, d, n_ch)(x_blocked, table_wide)
    return out.reshape(b0, s, d)
